# baseline (speedup 1.0000x reference)
import dataclasses
import functools

import jax
import jax.numpy as jnp
from jax import lax
from jax.experimental import pallas as pl
from jax.experimental.pallas import tpu as pltpu
from jax.experimental.pallas import tpu_sc as plsc

N_NODES = 50000
N_EDGES = 800000
N_CLASSES = 47
N_LAYERS = 50
ALPHA = 0.9

NTILES = 32
NPT = 1568
NPAD = NTILES * NPT
CP = 48

F = 2048
STAGE = 2 * F
CH1 = 2000
CH3 = 2048
CH4 = 128
CAP = 800768

_MESH = plsc.VectorSubcoreMesh(core_axis_name="c", subcore_axis_name="s")

_CP = pltpu.CompilerParams()
for _f, _v in (("needs_layout_passes", False), ("use_tc_tiling_on_sc", False)):
    if _f in pltpu.CompilerParams.__dataclass_fields__:
        _CP = dataclasses.replace(_CP, **{_f: _v})


def _wid():
    return lax.axis_index("s") * 2 + lax.axis_index("c")


def _k1_body(rowh, colh, orows, ocol, ocnt, odeg,
             rowc, colc, stage_r, stage_c, hist, degl, cntv):
    t = _wid()
    lo = pl.multiple_of(t * NPT, NPT)
    ebase = pl.multiple_of(t * CAP, F)
    iota = lax.iota(jnp.int32, 16)
    ones = jnp.ones((16,), jnp.float32)
    zf = jnp.zeros((16,), jnp.float32)
    zi = jnp.zeros((16,), jnp.int32)

    def zero_hist(j, c):
        for lane in range(16):
            hist[lane, pl.ds(j * 16, 16)] = zf
        return c
    lax.fori_loop(0, NPT // 16, zero_hist, 0)

    def zero_stage(j, c):
        stage_r[pl.ds(j * 16, 16)] = zi
        stage_c[pl.ds(j * 16, 16)] = zi
        return c
    lax.fori_loop(0, STAGE // 16, zero_stage, 0)

    def chunk_body(ci, carry):
        cnt, flushed = carry
        pltpu.sync_copy(rowh.at[pl.ds(ci * CH1, CH1)], rowc)
        pltpu.sync_copy(colh.at[pl.ds(ci * CH1, CH1)], colc)

        def vec_body(v, cnt):
            c16 = colc[pl.ds(v * 16, 16)]
            r16 = rowc[pl.ds(v * 16, 16)]
            m = (c16 >= lo) & (c16 < lo + NPT)
            mi = m.astype(jnp.int32)
            pos = cnt + plsc.cumsum(mi) - 1
            sidx = lax.rem(pos, STAGE)
            cl16 = c16 - lo
            plsc.store_scatter(stage_r, [sidx], r16, mask=m)
            plsc.store_scatter(stage_c, [sidx], cl16, mask=m)
            plsc.addupdate_scatter(hist, [iota, cl16], ones, mask=m)
            return cnt + jnp.sum(mi)

        cnt = lax.fori_loop(0, CH1 // 16, vec_body, cnt)

        def do_flush(fl):
            half = pl.multiple_of(lax.rem(fl, STAGE), F)
            dst = pl.multiple_of(ebase + fl, F)
            pltpu.sync_copy(stage_r.at[pl.ds(half, F)], orows.at[pl.ds(dst, F)])
            pltpu.sync_copy(stage_c.at[pl.ds(half, F)], ocol.at[pl.ds(dst, F)])
            return fl + F

        flushed = lax.cond(cnt - flushed >= F, do_flush, lambda fl: fl, flushed)
        return (cnt, flushed)

    cnt, flushed = lax.fori_loop(0, N_EDGES // CH1, chunk_body,
                                 (jnp.int32(0), jnp.int32(0)))

    half = pl.multiple_of(lax.rem(flushed, STAGE), F)
    dst = pl.multiple_of(ebase + flushed, F)
    pltpu.sync_copy(stage_r.at[pl.ds(half, F)], orows.at[pl.ds(dst, F)])
    pltpu.sync_copy(stage_c.at[pl.ds(half, F)], ocol.at[pl.ds(dst, F)])

    cntv[...] = jnp.full((16,), cnt, jnp.int32)
    pltpu.sync_copy(cntv, ocnt.at[pl.ds(pl.multiple_of(t * 16, 16), 16)])

    def red_body(j, c):
        acc = jnp.zeros((16,), jnp.float32)
        for lane in range(16):
            acc = acc + hist[lane, pl.ds(j * 16, 16)]
        degl[pl.ds(j * 16, 16)] = acc
        return c
    lax.fori_loop(0, NPT // 16, red_body, 0)
    pltpu.sync_copy(degl, odeg.at[pl.ds(lo, NPT)])


_k1 = functools.partial(
    pl.kernel,
    out_type=(
        jax.ShapeDtypeStruct((NTILES * CAP,), jnp.int32),
        jax.ShapeDtypeStruct((NTILES * CAP,), jnp.int32),
        jax.ShapeDtypeStruct((NTILES * 16,), jnp.int32),
        jax.ShapeDtypeStruct((NPAD,), jnp.float32),
    ),
    mesh=_MESH,
    scratch_types=[
        pltpu.VMEM((CH1,), jnp.int32),
        pltpu.VMEM((CH1,), jnp.int32),
        pltpu.VMEM((STAGE,), jnp.int32),
        pltpu.VMEM((STAGE,), jnp.int32),
        pltpu.VMEM((16, NPT), jnp.float32),
        pltpu.VMEM((NPT,), jnp.float32),
        pltpu.VMEM((16,), jnp.int32),
    ],
    compiler_params=_CP,
)(_k1_body)


def _rsqrt_body(deg_ref, dis_ref):
    d = deg_ref[...]
    dis_ref[...] = jnp.where(d > 0.0, lax.rsqrt(d), 0.0)


def _init_body(y_ref, m_ref, out0_ref, res_ref):
    o = y_ref[...] * m_ref[...]
    out0_ref[...] = o
    res_ref[...] = o * jnp.float32(1.0 - ALPHA)


def _k3_body(rowsh, colh, cnth, dish, onorm, disv, rowc, colc, normc, cntv):
    t = _wid()
    lo = pl.multiple_of(t * NPT, NPT)
    ebase = pl.multiple_of(t * CAP, F)
    pltpu.sync_copy(dish, disv)
    pltpu.sync_copy(cnth.at[pl.ds(pl.multiple_of(t * 16, 16), 16)], cntv)
    n = jnp.max(cntv[...])
    nch = (n + CH3 - 1) // CH3
    iota = lax.iota(jnp.int32, 16)
    alpha = jnp.float32(ALPHA)

    def ch_body(ci, c):
        base = pl.multiple_of(ci * CH3, CH3)
        pltpu.sync_copy(rowsh.at[pl.ds(ebase + base, CH3)], rowc)
        pltpu.sync_copy(colh.at[pl.ds(ebase + base, CH3)], colc)

        def vb(v, cc):
            r16 = rowc[pl.ds(v * 16, 16)]
            c16 = colc[pl.ds(v * 16, 16)] + lo
            m = (base + v * 16 + iota) < n
            dr = plsc.load_gather(disv, [r16], mask=m)
            dc = plsc.load_gather(disv, [c16], mask=m)
            normc[pl.ds(v * 16, 16)] = dr * dc * alpha
            return cc
        lax.fori_loop(0, CH3 // 16, vb, 0)
        pltpu.sync_copy(normc, onorm.at[pl.ds(ebase + base, CH3)])
        return c

    lax.fori_loop(0, nch, ch_body, 0)


_k3 = functools.partial(
    pl.kernel,
    out_type=jax.ShapeDtypeStruct((NTILES * CAP,), jnp.float32),
    mesh=_MESH,
    scratch_types=[
        pltpu.VMEM((NPAD,), jnp.float32),
        pltpu.VMEM((CH3,), jnp.int32),
        pltpu.VMEM((CH3,), jnp.int32),
        pltpu.VMEM((CH3,), jnp.float32),
        pltpu.VMEM((16,), jnp.int32),
    ],
    compiler_params=_CP,
)(_k3_body)


S4 = 768
G4 = S4 // 128
BK4 = 224


def _k4_body(outh, resh, rowsh, colh, normh, cnth, onew,
             acc, rowbuf, colbuf, normbuf, msgs, cntv, sem):
    t = _wid()
    lo = pl.multiple_of(t * NPT, NPT)
    ebase = pl.multiple_of(t * CAP, F)
    iota = lax.iota(jnp.int32, 16)
    iots = [iota, iota + 16, iota + 32]
    pltpu.sync_copy(resh.at[pl.ds(lo, NPT)], acc)
    pltpu.sync_copy(cnth.at[pl.ds(pl.multiple_of(t * 16, 16), 16)], cntv)
    n = jnp.max(cntv[...])
    nsc = (n + S4 - 1) // S4
    zf = jnp.zeros((16,), jnp.float32)

    def sc_body(si, c):
        base = pl.multiple_of(si * S4, S4)
        cps = [pltpu.async_copy(normh.at[pl.ds(ebase + base, S4)], normbuf, sem)]
        for k in range(G4):
            cps.append(pltpu.async_copy(
                rowsh.at[pl.ds(ebase + base + k * 128, 128)], rowbuf.at[k], sem))
            cps.append(pltpu.async_copy(
                colh.at[pl.ds(ebase + base + k * 128, 128)], colbuf.at[k], sem))
        for cp in cps:
            cp.wait()
        gps = [pltpu.async_copy(outh.at[rowbuf.at[k]], msgs.at[k], sem)
               for k in range(G4)]
        for gp in gps:
            gp.wait()
        ne = n - base

        for kk in range(G4):
            nek16 = jnp.full((16,), ne - kk * 128, jnp.int32)
            ksp = jnp.full((16,), kk, jnp.int32)

            def ebq(j, cc, _kk=kk, _ksp=ksp, _nek16=nek16):
                for u in range(4):
                    e2 = j * 4 + u
                    e2sp = jnp.full((16,), e2, jnp.int32)
                    esp = e2sp + _kk * 128
                    nrm = plsc.load_gather(normbuf, [esp])
                    nrm = jnp.where(e2sp < _nek16, nrm, zf)
                    cl = plsc.load_gather(colbuf, [_ksp, e2sp])
                    for k in range(3):
                        v = msgs[_kk, e2, pl.ds(k * 16, 16)] * nrm
                        plsc.addupdate_scatter(acc, [cl, iots[k]], v)
                return cc
            lax.fori_loop(0, 32, ebq, 0)
        return c

    lax.fori_loop(0, nsc, sc_body, 0)

    def cb(j, c):
        for k in range(3):
            x = acc[j, pl.ds(k * 16, 16)]
            acc[j, pl.ds(k * 16, 16)] = jnp.minimum(jnp.maximum(x, 0.0), 1.0)
        return c
    lax.fori_loop(0, NPT, cb, 0)
    pltpu.sync_copy(acc, onew.at[pl.ds(lo, NPT)])


_k4 = functools.partial(
    pl.kernel,
    out_type=jax.ShapeDtypeStruct((NPAD, CP), jnp.float32),
    mesh=_MESH,
    scratch_types=[
        pltpu.VMEM((NPT, CP), jnp.float32),
        pltpu.VMEM((G4, 128), jnp.int32),
        pltpu.VMEM((G4, 128), jnp.int32),
        pltpu.VMEM((S4,), jnp.float32),
        pltpu.VMEM((G4, 128, CP), jnp.float32),
        pltpu.VMEM((16,), jnp.int32),
        pltpu.SemaphoreType.DMA,
    ],
    compiler_params=_CP,
)(_k4_body)


def kernel(x, edge_index, y, train_mask):
    del x
    row = edge_index[0]
    col = edge_index[1]

    rows_s, col_s, counts, deg = _k1(row, col)

    dis2 = pl.pallas_call(
        _rsqrt_body,
        out_shape=jax.ShapeDtypeStruct((NPAD // 128, 128), jnp.float32),
    )(deg.reshape(NPAD // 128, 128))

    ypad = jnp.pad(y, ((0, NPAD - N_NODES), (0, CP - N_CLASSES)))
    maskp = jnp.pad(train_mask.astype(jnp.float32),
                    (0, NPAD - N_NODES)).reshape(NPAD, 1)
    out0, res = pl.pallas_call(
        _init_body,
        grid=(NPAD // 512,),
        in_specs=[
            pl.BlockSpec((512, CP), lambda i: (i, 0)),
            pl.BlockSpec((512, 1), lambda i: (i, 0)),
        ],
        out_specs=[
            pl.BlockSpec((512, CP), lambda i: (i, 0)),
            pl.BlockSpec((512, CP), lambda i: (i, 0)),
        ],
        out_shape=[jax.ShapeDtypeStruct((NPAD, CP), jnp.float32)] * 2,
    )(ypad, maskp)

    normv = _k3(rows_s, col_s, counts, dis2.reshape(NPAD))

    out = lax.fori_loop(
        0, N_LAYERS,
        lambda i, o: _k4(o, res, rows_s, col_s, normv, counts),
        out0,
    )
    return out[:N_NODES, :N_CLASSES]


# speedup vs baseline: 1.1094x; 1.1094x over previous
import dataclasses
import functools

import jax
import jax.numpy as jnp
from jax import lax
from jax.experimental import pallas as pl
from jax.experimental.pallas import tpu as pltpu
from jax.experimental.pallas import tpu_sc as plsc

N_NODES = 50000
N_EDGES = 800000
N_CLASSES = 47
N_LAYERS = 50
ALPHA = 0.9

NTILES = 32
NPT = 1568
NPAD = NTILES * NPT
CP = 48

F = 2048
STAGE = 2 * F
CH1 = 2000
CH3 = 2048
CH4 = 128
CAP = 802816

_MESH = plsc.VectorSubcoreMesh(core_axis_name="c", subcore_axis_name="s")

_CP = pltpu.CompilerParams()
for _f, _v in (("needs_layout_passes", False), ("use_tc_tiling_on_sc", False)):
    if _f in pltpu.CompilerParams.__dataclass_fields__:
        _CP = dataclasses.replace(_CP, **{_f: _v})


def _wid():
    return lax.axis_index("s") * 2 + lax.axis_index("c")


def _k1_body(rowh, colh, orows, ocol, ocnt, odeg,
             rowc, colc, stage_r, stage_c, hist, degl, cntv):
    t = _wid()
    lo = pl.multiple_of(t * NPT, NPT)
    ebase = pl.multiple_of(t * CAP, F)
    iota = lax.iota(jnp.int32, 16)
    ones = jnp.ones((16,), jnp.float32)
    zf = jnp.zeros((16,), jnp.float32)
    zi = jnp.zeros((16,), jnp.int32)

    def zero_hist(j, c):
        for lane in range(16):
            hist[lane, pl.ds(j * 16, 16)] = zf
        return c
    lax.fori_loop(0, NPT // 16, zero_hist, 0)

    def zero_stage(j, c):
        stage_r[pl.ds(j * 16, 16)] = zi
        stage_c[pl.ds(j * 16, 16)] = zi
        return c
    lax.fori_loop(0, STAGE // 16, zero_stage, 0)

    def chunk_body(ci, carry):
        cnt, flushed = carry
        pltpu.sync_copy(rowh.at[pl.ds(ci * CH1, CH1)], rowc)
        pltpu.sync_copy(colh.at[pl.ds(ci * CH1, CH1)], colc)

        def vec_body(v, cnt):
            c16 = colc[pl.ds(v * 16, 16)]
            r16 = rowc[pl.ds(v * 16, 16)]
            m = (c16 >= lo) & (c16 < lo + NPT)
            mi = m.astype(jnp.int32)
            pos = cnt + plsc.cumsum(mi) - 1
            sidx = lax.rem(pos, STAGE)
            cl16 = c16 - lo
            plsc.store_scatter(stage_r, [sidx], r16, mask=m)
            plsc.store_scatter(stage_c, [sidx], cl16, mask=m)
            plsc.addupdate_scatter(hist, [iota, cl16], ones, mask=m)
            return cnt + jnp.sum(mi)

        cnt = lax.fori_loop(0, CH1 // 16, vec_body, cnt)

        def do_flush(fl):
            half = pl.multiple_of(lax.rem(fl, STAGE), F)
            dst = pl.multiple_of(ebase + fl, F)
            pltpu.sync_copy(stage_r.at[pl.ds(half, F)], orows.at[pl.ds(dst, F)])
            pltpu.sync_copy(stage_c.at[pl.ds(half, F)], ocol.at[pl.ds(dst, F)])
            return fl + F

        flushed = lax.cond(cnt - flushed >= F, do_flush, lambda fl: fl, flushed)
        return (cnt, flushed)

    cnt, flushed = lax.fori_loop(0, N_EDGES // CH1, chunk_body,
                                 (jnp.int32(0), jnp.int32(0)))

    half = pl.multiple_of(lax.rem(flushed, STAGE), F)
    dst = pl.multiple_of(ebase + flushed, F)
    pltpu.sync_copy(stage_r.at[pl.ds(half, F)], orows.at[pl.ds(dst, F)])
    pltpu.sync_copy(stage_c.at[pl.ds(half, F)], ocol.at[pl.ds(dst, F)])
    half2 = pl.multiple_of(F - half, F)
    dst2 = pl.multiple_of(dst + F, F)
    pltpu.sync_copy(stage_r.at[pl.ds(half2, F)], orows.at[pl.ds(dst2, F)])
    pltpu.sync_copy(stage_c.at[pl.ds(half2, F)], ocol.at[pl.ds(dst2, F)])

    cntv[...] = jnp.full((16,), cnt, jnp.int32)
    pltpu.sync_copy(cntv, ocnt.at[pl.ds(pl.multiple_of(t * 16, 16), 16)])

    def red_body(j, c):
        acc = jnp.zeros((16,), jnp.float32)
        for lane in range(16):
            acc = acc + hist[lane, pl.ds(j * 16, 16)]
        degl[pl.ds(j * 16, 16)] = acc
        return c
    lax.fori_loop(0, NPT // 16, red_body, 0)
    pltpu.sync_copy(degl, odeg.at[pl.ds(lo, NPT)])


_k1 = functools.partial(
    pl.kernel,
    out_type=(
        jax.ShapeDtypeStruct((NTILES * CAP,), jnp.int32),
        jax.ShapeDtypeStruct((NTILES * CAP,), jnp.int32),
        jax.ShapeDtypeStruct((NTILES * 16,), jnp.int32),
        jax.ShapeDtypeStruct((NPAD,), jnp.float32),
    ),
    mesh=_MESH,
    scratch_types=[
        pltpu.VMEM((CH1,), jnp.int32),
        pltpu.VMEM((CH1,), jnp.int32),
        pltpu.VMEM((STAGE,), jnp.int32),
        pltpu.VMEM((STAGE,), jnp.int32),
        pltpu.VMEM((16, NPT), jnp.float32),
        pltpu.VMEM((NPT,), jnp.float32),
        pltpu.VMEM((16,), jnp.int32),
    ],
    compiler_params=_CP,
)(_k1_body)


def _rsqrt_body(deg_ref, dis_ref):
    d = deg_ref[...]
    dis_ref[...] = jnp.where(d > 0.0, lax.rsqrt(d), 0.0)


def _init_body(y_ref, m_ref, out0_ref, res_ref):
    o = y_ref[...] * m_ref[...]
    out0_ref[...] = o
    res_ref[...] = o * jnp.float32(1.0 - ALPHA)


def _k3_body(rowsh, colh, cnth, dish, onorm, disv, rowc, colc, normc, cntv):
    t = _wid()
    lo = pl.multiple_of(t * NPT, NPT)
    ebase = pl.multiple_of(t * CAP, F)
    pltpu.sync_copy(dish, disv)
    pltpu.sync_copy(cnth.at[pl.ds(pl.multiple_of(t * 16, 16), 16)], cntv)
    n = jnp.max(cntv[...])
    nch = (n + CH3 - 1) // CH3
    iota = lax.iota(jnp.int32, 16)
    alpha = jnp.float32(ALPHA)

    def ch_body(ci, c):
        base = pl.multiple_of(ci * CH3, CH3)
        pltpu.sync_copy(rowsh.at[pl.ds(ebase + base, CH3)], rowc)
        pltpu.sync_copy(colh.at[pl.ds(ebase + base, CH3)], colc)

        def vb(v, cc):
            r16 = rowc[pl.ds(v * 16, 16)]
            c16 = colc[pl.ds(v * 16, 16)] + lo
            m = (base + v * 16 + iota) < n
            dr = plsc.load_gather(disv, [r16], mask=m)
            dc = plsc.load_gather(disv, [c16], mask=m)
            normc[pl.ds(v * 16, 16)] = dr * dc * alpha
            return cc
        lax.fori_loop(0, CH3 // 16, vb, 0)
        pltpu.sync_copy(normc, onorm.at[pl.ds(ebase + base, CH3)])
        return c

    lax.fori_loop(0, nch, ch_body, 0)


_k3 = functools.partial(
    pl.kernel,
    out_type=jax.ShapeDtypeStruct((NTILES * CAP,), jnp.float32),
    mesh=_MESH,
    scratch_types=[
        pltpu.VMEM((NPAD,), jnp.float32),
        pltpu.VMEM((CH3,), jnp.int32),
        pltpu.VMEM((CH3,), jnp.int32),
        pltpu.VMEM((CH3,), jnp.float32),
        pltpu.VMEM((16,), jnp.int32),
    ],
    compiler_params=_CP,
)(_k3_body)


S4 = 384
G4 = S4 // 128


def _k4_body(outh, resh, rowsh, colh, normh, cnth, onew,
             acc, rowb0, rowb1, colb0, colb1, nrmb0, nrmb1, msg0, msg1,
             cntv, semE0, semE1, semG0, semG1):
    t = _wid()
    lo = pl.multiple_of(t * NPT, NPT)
    ebase = pl.multiple_of(t * CAP, F)
    iota = lax.iota(jnp.int32, 16)
    iots = [iota, iota + 16, iota + 32]
    zf = jnp.zeros((16,), jnp.float32)
    pltpu.sync_copy(resh.at[pl.ds(lo, NPT)], acc)
    pltpu.sync_copy(cnth.at[pl.ds(pl.multiple_of(t * 16, 16), 16)], cntv)
    n = jnp.max(cntv[...])
    nsc = (n + S4 - 1) // S4
    lastb = jnp.maximum(nsc - 1, 0) * S4

    rowb = [rowb0, rowb1]
    colb = [colb0, colb1]
    nrmb = [nrmb0, nrmb1]
    msg = [msg0, msg1]
    semE = [semE0, semE1]
    semG = [semG0, semG1]

    def basei(i):
        return pl.multiple_of(jnp.minimum(i * S4, lastb), 128)

    def issue_e(b, i):
        base = ebase + basei(i)
        pltpu.async_copy(normh.at[pl.ds(base, S4)], nrmb[b], semE[b])
        for k in range(G4):
            pltpu.async_copy(rowsh.at[pl.ds(base + k * 128, 128)],
                             rowb[b].at[k], semE[b])
            pltpu.async_copy(colh.at[pl.ds(base + k * 128, 128)],
                             colb[b].at[k], semE[b])

    def wait_e(b, i):
        base = ebase + basei(i)
        pltpu.make_async_copy(normh.at[pl.ds(base, S4)], nrmb[b], semE[b]).wait()
        for k in range(G4):
            pltpu.make_async_copy(rowsh.at[pl.ds(base + k * 128, 128)],
                                  rowb[b].at[k], semE[b]).wait()
            pltpu.make_async_copy(colh.at[pl.ds(base + k * 128, 128)],
                                  colb[b].at[k], semE[b]).wait()

    def issue_g(b):
        for k in range(G4):
            pltpu.async_copy(outh.at[rowb[b].at[k]], msg[b].at[k], semG[b])

    def wait_g(b):
        for k in range(G4):
            pltpu.make_async_copy(outh.at[rowb[b].at[k]], msg[b].at[k],
                                  semG[b]).wait()

    def compute(b, i):
        ne = n - i * S4
        for kk in range(G4):
            nek16 = jnp.full((16,), ne - kk * 128, jnp.int32)
            ksp = jnp.full((16,), kk, jnp.int32)

            def ebq(j, cc, _b=b, _kk=kk, _ksp=ksp, _nek16=nek16):
                for u in range(4):
                    e2 = j * 4 + u
                    e2sp = jnp.full((16,), e2, jnp.int32)
                    esp = e2sp + _kk * 128
                    nrm = plsc.load_gather(nrmb[_b], [esp])
                    nrm = jnp.where(e2sp < _nek16, nrm, zf)
                    cl = plsc.load_gather(colb[_b], [_ksp, e2sp])
                    for k in range(3):
                        v = msg[_b][_kk, e2, pl.ds(k * 16, 16)] * nrm
                        plsc.addupdate_scatter(acc, [cl, iots[k]], v)
                return cc
            lax.fori_loop(0, 32, ebq, 0)

    issue_e(0, 0)
    wait_e(0, 0)
    issue_g(0)
    issue_e(1, 1)
    npairs = (jnp.maximum(nsc, 2) + 1) // 2

    def pair_body(j, c):
        i0 = 2 * j
        wait_g(0)
        wait_e(1, i0 + 1)
        issue_g(1)
        compute(0, i0)
        issue_e(0, i0 + 2)
        wait_g(1)
        wait_e(0, i0 + 2)
        issue_g(0)
        compute(1, i0 + 1)
        issue_e(1, i0 + 3)
        return c

    lax.fori_loop(0, npairs, pair_body, 0)
    wait_g(0)
    wait_e(1, 2 * npairs + 1)

    def cb(j, c):
        for k in range(3):
            x = acc[j, pl.ds(k * 16, 16)]
            acc[j, pl.ds(k * 16, 16)] = jnp.minimum(jnp.maximum(x, 0.0), 1.0)
        return c
    lax.fori_loop(0, NPT, cb, 0)
    pltpu.sync_copy(acc, onew.at[pl.ds(lo, NPT)])


_k4 = functools.partial(
    pl.kernel,
    out_type=jax.ShapeDtypeStruct((NPAD, CP), jnp.float32),
    mesh=_MESH,
    scratch_types=[
        pltpu.VMEM((NPT, CP), jnp.float32),
        pltpu.VMEM((G4, 128), jnp.int32),
        pltpu.VMEM((G4, 128), jnp.int32),
        pltpu.VMEM((G4, 128), jnp.int32),
        pltpu.VMEM((G4, 128), jnp.int32),
        pltpu.VMEM((S4,), jnp.float32),
        pltpu.VMEM((S4,), jnp.float32),
        pltpu.VMEM((G4, 128, CP), jnp.float32),
        pltpu.VMEM((G4, 128, CP), jnp.float32),
        pltpu.VMEM((16,), jnp.int32),
        pltpu.SemaphoreType.DMA,
        pltpu.SemaphoreType.DMA,
        pltpu.SemaphoreType.DMA,
        pltpu.SemaphoreType.DMA,
    ],
    compiler_params=_CP,
)(_k4_body)


def kernel(x, edge_index, y, train_mask):
    del x
    row = edge_index[0]
    col = edge_index[1]

    rows_s, col_s, counts, deg = _k1(row, col)

    dis2 = pl.pallas_call(
        _rsqrt_body,
        out_shape=jax.ShapeDtypeStruct((NPAD // 128, 128), jnp.float32),
    )(deg.reshape(NPAD // 128, 128))

    ypad = jnp.pad(y, ((0, NPAD - N_NODES), (0, CP - N_CLASSES)))
    maskp = jnp.pad(train_mask.astype(jnp.float32),
                    (0, NPAD - N_NODES)).reshape(NPAD, 1)
    out0, res = pl.pallas_call(
        _init_body,
        grid=(NPAD // 512,),
        in_specs=[
            pl.BlockSpec((512, CP), lambda i: (i, 0)),
            pl.BlockSpec((512, 1), lambda i: (i, 0)),
        ],
        out_specs=[
            pl.BlockSpec((512, CP), lambda i: (i, 0)),
            pl.BlockSpec((512, CP), lambda i: (i, 0)),
        ],
        out_shape=[jax.ShapeDtypeStruct((NPAD, CP), jnp.float32)] * 2,
    )(ypad, maskp)

    normv = _k3(rows_s, col_s, counts, dis2.reshape(NPAD))

    out = lax.fori_loop(
        0, N_LAYERS,
        lambda i, o: _k4(o, res, rows_s, col_s, normv, counts),
        out0,
    )
    return out[:N_NODES, :N_CLASSES]
